# baseline (speedup 1.0000x reference)
import functools

import jax
import jax.numpy as jnp
from jax import lax
from jax.experimental import pallas as pl
from jax.experimental.pallas import tpu as pltpu
from jax.experimental.pallas import tpu_sc as plsc

N = 10000
E = 320000
D = 128
G = 128
NUM_LAYERS = 5
NUM_TASKS = 1

NC = 2
NS = 16
NW = NC * NS
EPW = E // NW
CHUNK = 64
CHUNKS = 160
EPW_PAD = CHUNKS * CHUNK
AGG_ROWS = 10240
ROWS_PER_SUB = AGG_ROWS // NS


NBUF = 5
LEAD = 3


def _agg_body(h_hbm, edges_hbm, zeros_hbm, out_hbm,
              pk_v, gidx, sidx, rows, agg_sh,
              i0, i1, i2, i3, i4,
              g0, g1, g2, g3, g4,
              s0, s1, s2, s3, s4, lsem):
    c = lax.axis_index("c")
    s = lax.axis_index("s")
    wid = s * NC + c
    isems = [i0, i1, i2, i3, i4]
    gsems = [g0, g1, g2, g3, g4]
    ssems = [s0, s1, s2, s3, s4]

    zd = pltpu.async_copy(zeros_hbm, agg_sh.at[pl.ds(s * ROWS_PER_SUB, ROWS_PER_SUB)], lsem)

    def fire_i(j, b):
        pltpu.async_copy(edges_hbm.at[wid * CHUNKS + j], pk_v.at[b], isems[b])

    def wait_i(j, b):
        pltpu.make_async_copy(edges_hbm.at[wid * CHUNKS + j], pk_v.at[b], isems[b]).wait()

    def unpack(b):
        for k in range(CHUNK // 16):
            p = pk_v[b, 0, pl.ds(k * 16, 16)]
            gidx[b, 0, pl.ds(k * 16, 16)] = lax.shift_right_logical(p, 14)
            sidx[b, 0, pl.ds(k * 16, 16)] = lax.bitwise_and(p, 16383)

    def fire_g(j, b):
            pltpu.async_copy(h_hbm.at[gidx.at[b, 0]], rows.at[b], gsems[b])

    def wait_g(j, b):
            pltpu.make_async_copy(h_hbm.at[gidx.at[b, 0]], rows.at[b], gsems[b]).wait()

    def fire_s(j, b):
            pltpu.async_copy(rows.at[b], agg_sh.at[sidx.at[b, 0]], ssems[b], add=True)

    def wait_s(j, b):
            pltpu.make_async_copy(rows.at[b], agg_sh.at[sidx.at[b, 0]], ssems[b]).wait()

    for j in range(NBUF):
        fire_i(j, j)
    zd.wait()
    plsc.subcore_barrier()
    for j in range(LEAD):
        wait_i(j, j)
        unpack(j)
        fire_g(j, j)

    def _group(i, _):
        for b in range(NBUF):
            j = i * NBUF + b

            @pl.when(j >= NBUF - LEAD)
            def _():
                wait_s(j - (NBUF - LEAD), (b - (NBUF - LEAD)) % NBUF)

            @pl.when(j + LEAD < CHUNKS)
            def _():
                wait_i(j + LEAD, (b + LEAD) % NBUF)
                unpack((b + LEAD) % NBUF)
                fire_g(j + LEAD, (b + LEAD) % NBUF)

            wait_g(j, b)
            fire_s(j, b)

            @pl.when(j + NBUF < CHUNKS)
            def _():
                fire_i(j + NBUF, b)
        return 0
    lax.fori_loop(0, CHUNKS // NBUF, _group, 0)

    for k in range(NBUF - LEAD):
        j = CHUNKS - (NBUF - LEAD) + k
        wait_s(j, j % NBUF)

    plsc.subcore_barrier()

    @pl.when(s < NS - 1)
    def _():
        pltpu.sync_copy(agg_sh.at[pl.ds(s * ROWS_PER_SUB, ROWS_PER_SUB)],
                        out_hbm.at[c, pl.ds(s * ROWS_PER_SUB, ROWS_PER_SUB)])

    @pl.when(s == NS - 1)
    def _():
        pltpu.sync_copy(agg_sh.at[pl.ds((NS - 1) * ROWS_PER_SUB, N - (NS - 1) * ROWS_PER_SUB)],
                        out_hbm.at[c, pl.ds((NS - 1) * ROWS_PER_SUB, N - (NS - 1) * ROWS_PER_SUB)])


_agg_call = pl.kernel(
    _agg_body,
    out_type=jax.ShapeDtypeStruct((NC, N, D), jnp.float32),
    mesh=plsc.VectorSubcoreMesh(core_axis_name="c", subcore_axis_name="s"),
    scratch_types=[
        pltpu.VMEM((NBUF, 1, CHUNK), jnp.int32),
        pltpu.VMEM((NBUF, 1, CHUNK), jnp.int32),
        pltpu.VMEM((NBUF, 1, CHUNK), jnp.int32),
        pltpu.VMEM((NBUF, CHUNK, D), jnp.float32),
        pltpu.VMEM_SHARED((AGG_ROWS, D), jnp.float32),
    ] + [pltpu.SemaphoreType.DMA] * (3 * NBUF + 1),
)


RB = 2000


def _layer_body(h_ref, a0_ref, a1_ref, w_ref, b_ref, o_ref, *, relu):
    t = h_ref[...] + a0_ref[...] + a1_ref[...]
    y = jnp.dot(t, w_ref[...], preferred_element_type=jnp.float32) + b_ref[...]
    if relu:
        y = jnp.maximum(y, 0.0)
    o_ref[...] = y


def _layer(h, a0, a1, w, b2, relu):
    return pl.pallas_call(
        functools.partial(_layer_body, relu=relu),
        grid=(N // RB,),
        in_specs=[
            pl.BlockSpec((RB, D), lambda i: (i, 0)),
            pl.BlockSpec((RB, D), lambda i: (i, 0)),
            pl.BlockSpec((RB, D), lambda i: (i, 0)),
            pl.BlockSpec((D, D), lambda i: (0, 0)),
            pl.BlockSpec((1, D), lambda i: (0, 0)),
        ],
        out_specs=pl.BlockSpec((RB, D), lambda i: (i, 0)),
        out_shape=jax.ShapeDtypeStruct((N, D), jnp.float32),
    )(h, a0, a1, w, b2)


def _pool_body(h_ref, bid_ref, wp_ref, bp_ref, o_ref, sums, cnts):
    i = pl.program_id(0)

    @pl.when(i == 0)
    def _():
        sums[...] = jnp.zeros_like(sums)
        cnts[...] = jnp.zeros_like(cnts)

    bid = bid_ref[...]
    gi = lax.broadcasted_iota(jnp.int32, (RB, G), 1)
    oh = (bid == gi).astype(jnp.float32)
    hb = h_ref[...]
    dn = (((0,), (0,)), ((), ()))
    sums[...] += lax.dot_general(oh, hb, dn, preferred_element_type=jnp.float32)
    cnts[...] += lax.dot_general(oh, jnp.ones_like(hb), dn,
                                 preferred_element_type=jnp.float32)

    @pl.when(i == pl.num_programs(0) - 1)
    def _():
        pooled = sums[...] / jnp.maximum(cnts[...], 1.0)
        o_ref[...] = jnp.dot(pooled, wp_ref[...],
                             preferred_element_type=jnp.float32) + bp_ref[...]


def _pool(h, bid2, wp_pad, bp_pad):
    return pl.pallas_call(
        _pool_body,
        grid=(N // RB,),
        in_specs=[
            pl.BlockSpec((RB, D), lambda i: (i, 0)),
            pl.BlockSpec((RB, 1), lambda i: (i, 0)),
            pl.BlockSpec((D, D), lambda i: (0, 0)),
            pl.BlockSpec((1, D), lambda i: (0, 0)),
        ],
        out_specs=pl.BlockSpec((G, D), lambda i: (0, 0)),
        out_shape=jax.ShapeDtypeStruct((G, D), jnp.float32),
        scratch_shapes=[
            pltpu.VMEM((G, D), jnp.float32),
            pltpu.VMEM((G, D), jnp.float32),
        ],
    )(h, bid2, wp_pad, bp_pad)


def kernel(x, edge_index, batch_ids, alpha_adv, W, b, Wp, bp):
    src = edge_index[0].astype(jnp.int32)
    dst = edge_index[1].astype(jnp.int32)
    pk = jnp.sort((src << 14) | dst).reshape(NW, EPW)
    pad = EPW_PAD - EPW
    edges_p = jnp.pad(pk, ((0, 0), (0, pad)), constant_values=N).reshape(
        NW * CHUNKS, 1, CHUNK)

    zeros_hbm = jnp.zeros((ROWS_PER_SUB, D), jnp.float32)
    bid2 = batch_ids.astype(jnp.int32).reshape(N, 1)
    wp_pad = jnp.pad(Wp.astype(jnp.float32), ((0, 0), (0, D - NUM_TASKS)))
    bp_pad = jnp.pad(bp.astype(jnp.float32).reshape(1, NUM_TASKS),
                     ((0, 0), (0, D - NUM_TASKS)))

    h = x
    for l in range(NUM_LAYERS):
        agg = _agg_call(h, edges_p, zeros_hbm)
        h = _layer(h, agg[0], agg[1], W[l], b[l].reshape(1, D),
                   relu=(l < NUM_LAYERS - 1))

    out = _pool(h, bid2, wp_pad, bp_pad)
    return out[:, :NUM_TASKS]


# speedup vs baseline: 1.3725x; 1.3725x over previous
import functools

import jax
import jax.numpy as jnp
from jax import lax
from jax.experimental import pallas as pl
from jax.experimental.pallas import tpu as pltpu
from jax.experimental.pallas import tpu_sc as plsc

N = 10000
E = 320000
D = 128
G = 128
NUM_LAYERS = 5
NUM_TASKS = 1

NC = 2
NS = 16
NW = NC * NS
EPW = E // NW
CHUNK = 64
CHUNKS = 160
EPW_PAD = CHUNKS * CHUNK
AGG_ROWS = 10240
ROWS_PER_SUB = AGG_ROWS // NS


NBUF = 5
LEAD = 3


def _agg_body(h_hbm, edges_hbm, zeros_hbm, out_hbm,
              pk_v, gidx, sidx, rows, agg_sh,
              i0, i1, i2, i3, i4,
              g0, g1, g2, g3, g4,
              s0, s1, s2, s3, s4, lsem):
    c = lax.axis_index("c")
    s = lax.axis_index("s")
    wid = s * NC + c
    isems = [i0, i1, i2, i3, i4]
    gsems = [g0, g1, g2, g3, g4]
    ssems = [s0, s1, s2, s3, s4]

    zd = pltpu.async_copy(zeros_hbm, agg_sh.at[pl.ds(s * ROWS_PER_SUB, ROWS_PER_SUB)], lsem)

    def fire_i(j, b):
        pltpu.async_copy(edges_hbm.at[wid * CHUNKS + j], pk_v.at[b], isems[b])

    def wait_i(j, b):
        pltpu.make_async_copy(edges_hbm.at[wid * CHUNKS + j], pk_v.at[b], isems[b]).wait()

    def unpack(b):
        for k in range(CHUNK // 16):
            p = pk_v[b, 0, pl.ds(k * 16, 16)]
            gidx[b, 0, pl.ds(k * 16, 16)] = lax.shift_right_logical(p, 14)
            sidx[b, 0, pl.ds(k * 16, 16)] = lax.bitwise_and(p, 16383)

    def fire_g(j, b):
            pltpu.async_copy(h_hbm.at[gidx.at[b, 0]], rows.at[b], gsems[b])

    def wait_g(j, b):
            pltpu.make_async_copy(h_hbm.at[gidx.at[b, 0]], rows.at[b], gsems[b]).wait()

    def fire_s(j, b):
            pltpu.async_copy(rows.at[b], agg_sh.at[sidx.at[b, 0]], ssems[b], add=True)

    def wait_s(j, b):
            pltpu.make_async_copy(rows.at[b], agg_sh.at[sidx.at[b, 0]], ssems[b]).wait()

    for j in range(NBUF):
        fire_i(j, j)
    zd.wait()
    plsc.subcore_barrier()
    for j in range(LEAD):
        wait_i(j, j)
        unpack(j)
        fire_g(j, j)

    def _group(i, _):
        for b in range(NBUF):
            j = i * NBUF + b

            @pl.when(j >= NBUF - LEAD)
            def _():
                wait_s(j - (NBUF - LEAD), (b - (NBUF - LEAD)) % NBUF)

            @pl.when(j + LEAD < CHUNKS)
            def _():
                wait_i(j + LEAD, (b + LEAD) % NBUF)
                unpack((b + LEAD) % NBUF)
                fire_g(j + LEAD, (b + LEAD) % NBUF)

            wait_g(j, b)
            fire_s(j, b)

            @pl.when(j + NBUF < CHUNKS)
            def _():
                fire_i(j + NBUF, b)
        return 0
    lax.fori_loop(0, CHUNKS // NBUF, _group, 0)

    for k in range(NBUF - LEAD):
        j = CHUNKS - (NBUF - LEAD) + k
        wait_s(j, j % NBUF)

    plsc.subcore_barrier()

    @pl.when(s < NS - 1)
    def _():
        pltpu.sync_copy(agg_sh.at[pl.ds(s * ROWS_PER_SUB, ROWS_PER_SUB)],
                        out_hbm.at[c, pl.ds(s * ROWS_PER_SUB, ROWS_PER_SUB)])

    @pl.when(s == NS - 1)
    def _():
        pltpu.sync_copy(agg_sh.at[pl.ds((NS - 1) * ROWS_PER_SUB, N - (NS - 1) * ROWS_PER_SUB)],
                        out_hbm.at[c, pl.ds((NS - 1) * ROWS_PER_SUB, N - (NS - 1) * ROWS_PER_SUB)])


_agg_call = pl.kernel(
    _agg_body,
    out_type=jax.ShapeDtypeStruct((NC, N, D), jnp.float32),
    mesh=plsc.VectorSubcoreMesh(core_axis_name="c", subcore_axis_name="s"),
    scratch_types=[
        pltpu.VMEM((NBUF, 1, CHUNK), jnp.int32),
        pltpu.VMEM((NBUF, 1, CHUNK), jnp.int32),
        pltpu.VMEM((NBUF, 1, CHUNK), jnp.int32),
        pltpu.VMEM((NBUF, CHUNK, D), jnp.float32),
        pltpu.VMEM_SHARED((AGG_ROWS, D), jnp.float32),
    ] + [pltpu.SemaphoreType.DMA] * (3 * NBUF + 1),
)


RB = 2000


def _layer_body(h_ref, a0_ref, a1_ref, w_ref, b_ref, o_ref, *, relu):
    t = h_ref[...] + a0_ref[...] + a1_ref[...]
    y = jnp.dot(t, w_ref[...], preferred_element_type=jnp.float32) + b_ref[...]
    if relu:
        y = jnp.maximum(y, 0.0)
    o_ref[...] = y


def _layer(h, a0, a1, w, b2, relu):
    return pl.pallas_call(
        functools.partial(_layer_body, relu=relu),
        grid=(N // RB,),
        in_specs=[
            pl.BlockSpec((RB, D), lambda i: (i, 0)),
            pl.BlockSpec((RB, D), lambda i: (i, 0)),
            pl.BlockSpec((RB, D), lambda i: (i, 0)),
            pl.BlockSpec((D, D), lambda i: (0, 0)),
            pl.BlockSpec((1, D), lambda i: (0, 0)),
        ],
        out_specs=pl.BlockSpec((RB, D), lambda i: (i, 0)),
        out_shape=jax.ShapeDtypeStruct((N, D), jnp.float32),
    )(h, a0, a1, w, b2)


def _final_body(h_ref, a0_ref, a1_ref, w_ref, b_ref, bid_ref, wp_ref, bp_ref,
                o_ref, sums, cnts):
    i = pl.program_id(0)

    @pl.when(i == 0)
    def _():
        sums[...] = jnp.zeros_like(sums)
        cnts[...] = jnp.zeros_like(cnts)

    t = h_ref[...] + a0_ref[...] + a1_ref[...]
    y = jnp.dot(t, w_ref[...], preferred_element_type=jnp.float32) + b_ref[...]

    bid = bid_ref[...]
    gi = lax.broadcasted_iota(jnp.int32, (RB, G), 1)
    oh = (bid == gi).astype(jnp.float32)
    dn = (((0,), (0,)), ((), ()))
    sums[...] += lax.dot_general(oh, y, dn, preferred_element_type=jnp.float32)
    cnts[...] += lax.dot_general(oh, jnp.ones_like(y), dn,
                                 preferred_element_type=jnp.float32)

    @pl.when(i == pl.num_programs(0) - 1)
    def _():
        pooled = sums[...] / jnp.maximum(cnts[...], 1.0)
        o_ref[...] = jnp.dot(pooled, wp_ref[...],
                             preferred_element_type=jnp.float32) + bp_ref[...]


def _final(h, a0, a1, w, b2, bid2, wp_pad, bp_pad):
    return pl.pallas_call(
        _final_body,
        grid=(N // RB,),
        in_specs=[
            pl.BlockSpec((RB, D), lambda i: (i, 0)),
            pl.BlockSpec((RB, D), lambda i: (i, 0)),
            pl.BlockSpec((RB, D), lambda i: (i, 0)),
            pl.BlockSpec((D, D), lambda i: (0, 0)),
            pl.BlockSpec((1, D), lambda i: (0, 0)),
            pl.BlockSpec((RB, 1), lambda i: (i, 0)),
            pl.BlockSpec((D, D), lambda i: (0, 0)),
            pl.BlockSpec((1, D), lambda i: (0, 0)),
        ],
        out_specs=pl.BlockSpec((G, D), lambda i: (0, 0)),
        out_shape=jax.ShapeDtypeStruct((G, D), jnp.float32),
        scratch_shapes=[
            pltpu.VMEM((G, D), jnp.float32),
            pltpu.VMEM((G, D), jnp.float32),
        ],
    )(h, a0, a1, w, b2, bid2, wp_pad, bp_pad)


def kernel(x, edge_index, batch_ids, alpha_adv, W, b, Wp, bp):
    src = edge_index[0].astype(jnp.int32)
    dst = edge_index[1].astype(jnp.int32)
    pk = ((src << 14) | dst).reshape(NW, EPW)
    pad = EPW_PAD - EPW
    edges_p = jnp.pad(pk, ((0, 0), (0, pad)), constant_values=N).reshape(
        NW * CHUNKS, 1, CHUNK)

    zeros_hbm = jnp.zeros((ROWS_PER_SUB, D), jnp.float32)
    bid2 = batch_ids.astype(jnp.int32).reshape(N, 1)
    wp_pad = jnp.pad(Wp.astype(jnp.float32), ((0, 0), (0, D - NUM_TASKS)))
    bp_pad = jnp.pad(bp.astype(jnp.float32).reshape(1, NUM_TASKS),
                     ((0, 0), (0, D - NUM_TASKS)))

    h = x
    for l in range(NUM_LAYERS - 1):
        agg = _agg_call(h, edges_p, zeros_hbm)
        h = _layer(h, agg[0], agg[1], W[l], b[l].reshape(1, D), relu=True)

    agg = _agg_call(h, edges_p, zeros_hbm)
    out = _final(h, agg[0], agg[1], W[NUM_LAYERS - 1],
                 b[NUM_LAYERS - 1].reshape(1, D), bid2, wp_pad, bp_pad)
    return out[:, :NUM_TASKS]
